# baseline (speedup 1.0000x reference)
import jax
import jax.numpy as jnp
from jax import lax
from jax.experimental import pallas as pl
from jax.experimental.pallas import tpu as pltpu
from jax.experimental.pallas import tpu_sc as plsc

N_NODES = 10000
N_EDGES = 320000
D_FEAT = 128
D_EDGE = 16

NUM_CORES = 2
NUM_SUBCORES = 16
NUM_WORKERS = NUM_CORES * NUM_SUBCORES

NUM_BLKS = N_EDGES // 128
BLKS_PER_TILE = NUM_BLKS // NUM_WORKERS
TAIL_TILES = NUM_BLKS - BLKS_PER_TILE * NUM_WORKERS
NC = 13
N_CHUNKS = BLKS_PER_TILE // NC
CHUNK_EDGES = NC * 128
ACC_ROWS = 10240
ROWS_PER_TILE = ACC_ROWS // NUM_SUBCORES

BLK_E = 6400
BLK_R = BLK_E // 8


def _lin_body(ea_ref, o_ref):
    eye = jnp.eye(D_EDGE, dtype=jnp.float32)
    t = lax.dot_general(ea_ref[...], eye, (((0,), (0,)), ((), ())),
                        preferred_element_type=jnp.float32)
    t3 = t.reshape(BLK_R, 8, D_EDGE)
    o_ref[...] = jnp.concatenate([t3[:, k, :] for k in range(8)], axis=-1)


def _tc_linearize(ea_t):
    return pl.pallas_call(
        _lin_body,
        grid=(N_EDGES // BLK_E,),
        in_specs=[pl.BlockSpec((D_EDGE, BLK_E), lambda i: (0, i))],
        out_specs=pl.BlockSpec((BLK_R, 128), lambda i: (i, 0)),
        out_shape=jax.ShapeDtypeStruct((N_EDGES // 8, 128), jnp.float32),
    )(ea_t)


def _sc_scatter_body(ei3_hbm, ea_hbm, out_hbm, idx_v, rows_v, zbuf_v, acc_sh):
    c = lax.axis_index("c")
    s = lax.axis_index("s")
    wid = c * NUM_SUBCORES + s
    blk_lo = wid * BLKS_PER_TILE

    def _zero(i, carry):
        zbuf_v[i, :] = jnp.zeros((16,), jnp.float32)
        return carry

    lax.fori_loop(0, ROWS_PER_TILE, _zero, 0)
    pltpu.sync_copy(zbuf_v, acc_sh.at[pl.ds(s * ROWS_PER_TILE, ROWS_PER_TILE)])

    pltpu.sync_copy(ei3_hbm.at[pl.ds(blk_lo, BLKS_PER_TILE), 1],
                    idx_v.at[pl.ds(0, BLKS_PER_TILE)])

    @pl.when(wid < TAIL_TILES)
    def _tail_idx():
        pltpu.sync_copy(
            ei3_hbm.at[pl.ds(NUM_BLKS - TAIL_TILES + wid, 1), 1],
            idx_v.at[pl.ds(BLKS_PER_TILE, 1)],
        )

    plsc.subcore_barrier()

    def _chunk(k, carry):
        e0 = blk_lo * 128 + k * CHUNK_EDGES
        pltpu.sync_copy(ea_hbm.at[pl.ds(e0, CHUNK_EDGES)], rows_v)
        for j in range(NC):
            pltpu.sync_copy(rows_v.at[pl.ds(j * 128, 128)],
                            acc_sh.at[idx_v.at[k * NC + j]], add=True)
        return carry

    lax.fori_loop(0, N_CHUNKS, _chunk, 0)

    @pl.when(wid < TAIL_TILES)
    def _tail():
        e0 = (NUM_BLKS - TAIL_TILES + wid) * 128
        pltpu.sync_copy(ea_hbm.at[pl.ds(e0, 128)], rows_v.at[pl.ds(0, 128)])
        pltpu.sync_copy(rows_v.at[pl.ds(0, 128)],
                        acc_sh.at[idx_v.at[BLKS_PER_TILE]], add=True)

    plsc.subcore_barrier()

    pltpu.sync_copy(
        acc_sh.at[pl.ds(s * ROWS_PER_TILE, ROWS_PER_TILE)],
        out_hbm.at[c, pl.ds(s * ROWS_PER_TILE, ROWS_PER_TILE)],
    )


_sc_scatter = pl.kernel(
    _sc_scatter_body,
    out_type=jax.ShapeDtypeStruct((NUM_CORES, ACC_ROWS, D_EDGE), jnp.float32),
    mesh=plsc.VectorSubcoreMesh(core_axis_name="c", subcore_axis_name="s"),
    scratch_types=[
        pltpu.VMEM((BLKS_PER_TILE + 2, 128), jnp.int32),
        pltpu.VMEM((CHUNK_EDGES, D_EDGE), jnp.float32),
        pltpu.VMEM((ROWS_PER_TILE, D_EDGE), jnp.float32),
        pltpu.VMEM_SHARED((ACC_ROWS, D_EDGE), jnp.float32),
    ],
    compiler_params=pltpu.CompilerParams(use_tc_tiling_on_sc=False),
)


def _mlp_body(x_ref, p_ref, w_ref, b_ref, o_ref):
    agg = p_ref[0] + p_ref[1]
    wx = w_ref[:D_FEAT, :]
    we = w_ref[D_FEAT:, :]
    o_ref[...] = (
        jnp.dot(x_ref[...], wx, preferred_element_type=jnp.float32)
        + jnp.dot(agg, we, preferred_element_type=jnp.float32)
        + b_ref[...]
    )


BLOCK_N = 2000


def _tc_mlp(x, partials, W, b2):
    grid = (N_NODES // BLOCK_N,)
    return pl.pallas_call(
        _mlp_body,
        grid=grid,
        in_specs=[
            pl.BlockSpec((BLOCK_N, D_FEAT), lambda i: (i, 0)),
            pl.BlockSpec((NUM_CORES, BLOCK_N, D_EDGE), lambda i: (0, i, 0)),
            pl.BlockSpec((D_FEAT + D_EDGE, D_FEAT), lambda i: (0, 0)),
            pl.BlockSpec((1, D_FEAT), lambda i: (0, 0)),
        ],
        out_specs=pl.BlockSpec((BLOCK_N, D_FEAT), lambda i: (i, 0)),
        out_shape=jax.ShapeDtypeStruct((N_NODES, D_FEAT), jnp.float32),
    )(x, partials, W, b2)


@jax.jit
def kernel(x, edge_index, edge_attr, W, b):
    ei3 = edge_index.T.reshape(NUM_BLKS, 128, 2).transpose(0, 2, 1)
    ea_lin = _tc_linearize(edge_attr.T)
    ea_rows = ea_lin.reshape(N_EDGES, D_EDGE)
    partials = _sc_scatter(ei3, ea_rows)
    return _tc_mlp(x, partials, W, b.reshape(1, D_FEAT))


# speedup vs baseline: 1.0411x; 1.0411x over previous
import jax
import jax.numpy as jnp
from jax import lax
from jax.experimental import pallas as pl
from jax.experimental.pallas import tpu as pltpu
from jax.experimental.pallas import tpu_sc as plsc

N_NODES = 10000
N_EDGES = 320000
D_FEAT = 128
D_EDGE = 16

NUM_CORES = 2
NUM_SUBCORES = 16
NUM_WORKERS = NUM_CORES * NUM_SUBCORES

NUM_BLKS = N_EDGES // 128
BLKS_PER_TILE = NUM_BLKS // NUM_WORKERS
TAIL_TILES = NUM_BLKS - BLKS_PER_TILE * NUM_WORKERS
NC = 13
N_CHUNKS = BLKS_PER_TILE // NC
ACC_ROWS = 10240
ROWS_PER_TILE = ACC_ROWS // NUM_SUBCORES


def _sc_scatter_body(ei3_hbm, ea4_hbm, out_hbm, idx_v, valsT_v, rows_v, zbuf_v,
                     acc_sh):
    c = lax.axis_index("c")
    s = lax.axis_index("s")
    wid = c * NUM_SUBCORES + s
    blk_lo = wid * BLKS_PER_TILE

    def _zero(i, carry):
        zbuf_v[i, :] = jnp.zeros((16,), jnp.float32)
        return carry

    lax.fori_loop(0, ROWS_PER_TILE, _zero, 0)
    pltpu.sync_copy(zbuf_v, acc_sh.at[pl.ds(s * ROWS_PER_TILE, ROWS_PER_TILE)])

    pltpu.sync_copy(ei3_hbm.at[pl.ds(blk_lo, BLKS_PER_TILE), 1],
                    idx_v.at[pl.ds(0, BLKS_PER_TILE)])

    @pl.when(wid < TAIL_TILES)
    def _tail_idx():
        pltpu.sync_copy(
            ei3_hbm.at[pl.ds(NUM_BLKS - TAIL_TILES + wid, 1), 1],
            idx_v.at[pl.ds(BLKS_PER_TILE, 1)],
        )

    plsc.subcore_barrier()

    iota = lax.iota(jnp.int32, 16)

    def _transpose(n_blk, valsT, rows):
        def _grp(g, carry2):
            row = g // 8
            lane0 = (g % 8) * 16
            ridx = g * 16 + iota
            for f in range(16):
                v = valsT[f, row, pl.ds(lane0, 16)]
                plsc.store_scatter(rows, [ridx, jnp.full((16,), f, jnp.int32)], v)
            return carry2

        lax.fori_loop(0, n_blk * 8, _grp, 0)

    def _chunk(k, carry):
        blk0 = blk_lo + k * NC
        for tr in range(2):
            for r in range(8):
                pltpu.sync_copy(ea4_hbm.at[tr, pl.ds(blk0, NC), r],
                                valsT_v.at[tr * 8 + r])
        _transpose(NC, valsT_v, rows_v)
        for j in range(NC):
            pltpu.sync_copy(rows_v.at[pl.ds(j * 128, 128)],
                            acc_sh.at[idx_v.at[k * NC + j]], add=True)
        return carry

    lax.fori_loop(0, N_CHUNKS, _chunk, 0)

    @pl.when(wid < TAIL_TILES)
    def _tail():
        blk = NUM_BLKS - TAIL_TILES + wid
        for tr in range(2):
            for r in range(8):
                pltpu.sync_copy(ea4_hbm.at[tr, pl.ds(blk, 1), r],
                                valsT_v.at[tr * 8 + r, pl.ds(0, 1)])
        _transpose(1, valsT_v, rows_v)
        pltpu.sync_copy(rows_v.at[pl.ds(0, 128)],
                        acc_sh.at[idx_v.at[BLKS_PER_TILE]], add=True)

    plsc.subcore_barrier()

    pltpu.sync_copy(
        acc_sh.at[pl.ds(s * ROWS_PER_TILE, ROWS_PER_TILE)],
        out_hbm.at[c, pl.ds(s * ROWS_PER_TILE, ROWS_PER_TILE)],
    )


_sc_scatter = pl.kernel(
    _sc_scatter_body,
    out_type=jax.ShapeDtypeStruct((NUM_CORES, ACC_ROWS, D_EDGE), jnp.float32),
    mesh=plsc.VectorSubcoreMesh(core_axis_name="c", subcore_axis_name="s"),
    scratch_types=[
        pltpu.VMEM((BLKS_PER_TILE + 2, 128), jnp.int32),
        pltpu.VMEM((D_EDGE, NC, 128), jnp.float32),
        pltpu.VMEM((NC * 128, D_EDGE), jnp.float32),
        pltpu.VMEM((ROWS_PER_TILE, D_EDGE), jnp.float32),
        pltpu.VMEM_SHARED((ACC_ROWS, D_EDGE), jnp.float32),
    ],
    compiler_params=pltpu.CompilerParams(use_tc_tiling_on_sc=False,
                                         needs_layout_passes=False),
)


def _mlp_body(x_ref, p_ref, w_ref, b_ref, o_ref):
    agg = p_ref[0] + p_ref[1]
    wx = w_ref[:D_FEAT, :]
    we = w_ref[D_FEAT:, :]
    o_ref[...] = (
        jnp.dot(x_ref[...], wx, preferred_element_type=jnp.float32)
        + jnp.dot(agg, we, preferred_element_type=jnp.float32)
        + b_ref[...]
    )


BLOCK_N = 2000


def _tc_mlp(x, partials, W, b2):
    grid = (N_NODES // BLOCK_N,)
    return pl.pallas_call(
        _mlp_body,
        grid=grid,
        in_specs=[
            pl.BlockSpec((BLOCK_N, D_FEAT), lambda i: (i, 0)),
            pl.BlockSpec((NUM_CORES, BLOCK_N, D_EDGE), lambda i: (0, i, 0)),
            pl.BlockSpec((D_FEAT + D_EDGE, D_FEAT), lambda i: (0, 0)),
            pl.BlockSpec((1, D_FEAT), lambda i: (0, 0)),
        ],
        out_specs=pl.BlockSpec((BLOCK_N, D_FEAT), lambda i: (i, 0)),
        out_shape=jax.ShapeDtypeStruct((N_NODES, D_FEAT), jnp.float32),
    )(x, partials, W, b2)


@jax.jit
def kernel(x, edge_index, edge_attr, W, b):
    ea4 = edge_attr.T.reshape(2, 8, NUM_BLKS, 128).transpose(0, 2, 1, 3)
    ei3 = edge_index.T.reshape(NUM_BLKS, 128, 2).transpose(0, 2, 1)
    partials = _sc_scatter(ei3, ea4)
    return _tc_mlp(x, partials, W, b.reshape(1, D_FEAT))
